# baseline (speedup 1.0000x reference)
import functools

import jax
import jax.numpy as jnp
from jax import lax
from jax.experimental import pallas as pl
from jax.experimental.pallas import tpu as pltpu
from jax.experimental.pallas import tpu_sc as plsc

B, C, H, W = 16, 16, 128, 128
N = C * H * W
TOTAL = B * N
AC_LEN = 1000

NC, NS, LANES = 2, 16, 16
NW = NC * NS
SC_BATCHES = 4
SC_PER_WORKER = SC_BATCHES * N // NW
CHUNK = 16384
NCHUNK = SC_PER_WORKER // CHUNK
WORKERS_PER_BATCH = NW // SC_BATCHES


def _babylonian_sqrt(x):
    y = jnp.full(x.shape, jnp.float32(1.0))
    for _ in range(15):
        y = jnp.float32(0.5) * (y + x / y)
    return y


def _sc_body(noise_hbm, xt_hbm, ts_hbm, ac_hbm, out_hbm,
             ts_v, ac_v, nb0, xb0, ob0, nb1, xb1, ob1,
             si0, si1, so0, so1):
    wid = lax.axis_index("s") * NC + lax.axis_index("c")
    base = wid * SC_PER_WORKER
    b = wid // WORKERS_PER_BATCH

    pltpu.sync_copy(ts_hbm, ts_v)
    pltpu.sync_copy(ac_hbm, ac_v.at[pl.ds(0, AC_LEN)])
    ts_pair = ts_v[pl.ds(2 * b, LANES)]
    t_b = ts_pair[0] | ts_pair[1]
    a_scalar = ac_v[pl.ds(t_b, LANES)][0]
    a = jnp.full((LANES,), a_scalar, dtype=jnp.float32)
    sqrt_a = _babylonian_sqrt(a)
    sqrt_beta = _babylonian_sqrt(jnp.float32(1.0) - a)
    s1 = jnp.float32(1.0) / sqrt_a
    s2 = sqrt_beta / sqrt_a

    bufs = ((nb0, xb0, ob0, si0, so0), (nb1, xb1, ob1, si1, so1))
    in_handles = {}
    out_handles = {}

    def start_in(k):
        nb, xb, _, si, _ = bufs[k % 2]
        off = base + k * CHUNK
        h1 = pltpu.async_copy(noise_hbm.at[pl.ds(off, CHUNK)], nb, si)
        h2 = pltpu.async_copy(xt_hbm.at[pl.ds(off, CHUNK)], xb, si)
        in_handles[k] = (h1, h2)

    def compute(k):
        nb, xb, ob, _, _ = bufs[k % 2]

        @plsc.parallel_loop(jnp.int32(0), jnp.int32(CHUNK), jnp.int32(LANES),
                            unroll=8)
        def _(i):
            sl = pl.ds(i, LANES)
            ob[sl] = xb[sl] * s1 - nb[sl] * s2

    def start_out(k):
        _, _, ob, _, so = bufs[k % 2]
        off = base + k * CHUNK
        out_handles[k] = pltpu.async_copy(ob, out_hbm.at[pl.ds(off, CHUNK)], so)

    start_in(0)
    for k in range(NCHUNK):
        if k + 1 < NCHUNK:
            start_in(k + 1)
        for h in in_handles.pop(k):
            h.wait()
        if k >= 2:
            out_handles.pop(k - 2).wait()
        compute(k)
        start_out(k)
    for k in sorted(out_handles):
        out_handles[k].wait()


_sc_call = functools.partial(
    pl.kernel,
    out_type=jax.ShapeDtypeStruct((SC_BATCHES * N,), jnp.float32),
    mesh=plsc.VectorSubcoreMesh(
        core_axis_name="c", subcore_axis_name="s",
        num_cores=NC, num_subcores=NS),
    scratch_types=[
        pltpu.VMEM((2 * B,), jnp.int32),
        pltpu.VMEM((1024,), jnp.float32),
        pltpu.VMEM((CHUNK,), jnp.float32),
        pltpu.VMEM((CHUNK,), jnp.float32),
        pltpu.VMEM((CHUNK,), jnp.float32),
        pltpu.VMEM((CHUNK,), jnp.float32),
        pltpu.VMEM((CHUNK,), jnp.float32),
        pltpu.VMEM((CHUNK,), jnp.float32),
        pltpu.SemaphoreType.DMA,
        pltpu.SemaphoreType.DMA,
        pltpu.SemaphoreType.DMA,
        pltpu.SemaphoreType.DMA,
    ],
)(_sc_body)

NROW = N // 128
TROW = 512


def _tc_body(ts_ref, ac_ref, noise_ref, xt_ref, out_ref):
    b = pl.program_id(0) + SC_BATCHES
    t = ts_ref[2 * b] | ts_ref[2 * b + 1]
    a = ac_ref[t]
    s1 = lax.rsqrt(a)
    s2 = jnp.sqrt(jnp.float32(1.0) - a) * s1
    out_ref[...] = xt_ref[...] * s1 - noise_ref[...] * s2


def _tc_index_map(b, j):
    return (b + jnp.int32(SC_BATCHES), j, jnp.int32(0))


_tc_call = pl.pallas_call(
    _tc_body,
    grid=(B - SC_BATCHES, NROW // TROW),
    in_specs=[
        pl.BlockSpec((2 * B,), lambda b, j: (jnp.int32(0),),
                     memory_space=pltpu.SMEM),
        pl.BlockSpec((AC_LEN,), lambda b, j: (jnp.int32(0),),
                     memory_space=pltpu.SMEM),
        pl.BlockSpec((1, TROW, 128), _tc_index_map),
        pl.BlockSpec((1, TROW, 128), _tc_index_map),
    ],
    out_specs=pl.BlockSpec((1, TROW, 128), _tc_index_map),
    out_shape=jax.ShapeDtypeStruct((B, NROW, 128), jnp.float32),
)


def kernel(noise, xt, timestep, alphas_cumprod):
    noise3d = noise.reshape(B, NROW, 128)
    xt3d = xt.reshape(B, NROW, 128)
    ts32 = lax.bitcast_convert_type(timestep, jnp.int32).reshape(2 * B)
    ac = alphas_cumprod.astype(jnp.float32)

    sc_out = _sc_call(noise.reshape(TOTAL), xt.reshape(TOTAL), ts32, ac)
    tc_out = _tc_call(ts32, ac, noise3d, xt3d)
    out = lax.dynamic_update_slice(
        tc_out, sc_out.reshape(SC_BATCHES, NROW, 128), (0, 0, 0))
    return out.reshape(B, C, H, W)


# speedup vs baseline: 1.3708x; 1.3708x over previous
import functools

import jax
import jax.numpy as jnp
from jax import lax
from jax.experimental import pallas as pl
from jax.experimental.pallas import tpu as pltpu
from jax.experimental.pallas import tpu_sc as plsc

B, C, H, W = 16, 16, 128, 128
N = C * H * W
TOTAL = B * N
AC_LEN = 1000

NC, NS, LANES = 2, 16, 16
NW = NC * NS
SC_BATCHES = 16
SC_PER_WORKER = SC_BATCHES * N // NW
CHUNK = 16384
NCHUNK = SC_PER_WORKER // CHUNK
NGROUP = NCHUNK // 2
WORKERS_PER_BATCH = NW // SC_BATCHES


def _babylonian_sqrt(x):
    y = jnp.full(x.shape, jnp.float32(1.0))
    for _ in range(15):
        y = jnp.float32(0.5) * (y + x / y)
    return y


def _sc_body(noise_hbm, xt_hbm, ts_hbm, ac_hbm, out_hbm,
             ts_v, ac_v, nb0, xb0, ob0, nb1, xb1, ob1,
             si0, si1, so0, so1):
    wid = lax.axis_index("s") * NC + lax.axis_index("c")
    base = wid * SC_PER_WORKER
    b = wid // WORKERS_PER_BATCH

    pltpu.sync_copy(ts_hbm, ts_v)
    pltpu.sync_copy(ac_hbm, ac_v.at[pl.ds(0, AC_LEN)])
    ts_pair = ts_v[pl.ds(2 * b, LANES)]
    t_b = ts_pair[0] | ts_pair[1]
    a_scalar = ac_v[pl.ds(t_b, LANES)][0]
    a = jnp.full((LANES,), a_scalar, dtype=jnp.float32)
    sqrt_a = _babylonian_sqrt(a)
    sqrt_beta = _babylonian_sqrt(jnp.float32(1.0) - a)
    s1 = jnp.float32(1.0) / sqrt_a
    s2 = sqrt_beta / sqrt_a

    bufs = ((nb0, xb0, ob0, si0, so0), (nb1, xb1, ob1, si1, so1))

    def in_copies(k, slot):
        nb, xb, _, si, _ = bufs[slot]
        off = base + k * CHUNK
        return (
            pltpu.make_async_copy(noise_hbm.at[pl.ds(off, CHUNK)], nb, si),
            pltpu.make_async_copy(xt_hbm.at[pl.ds(off, CHUNK)], xb, si),
        )

    def out_copy(k, slot):
        _, _, ob, _, so = bufs[slot]
        off = base + k * CHUNK
        return pltpu.make_async_copy(ob, out_hbm.at[pl.ds(off, CHUNK)], so)

    def compute(slot):
        nb, xb, ob, _, _ = bufs[slot]

        @plsc.parallel_loop(jnp.int32(0), jnp.int32(CHUNK), jnp.int32(LANES),
                            unroll=8)
        def _(i):
            sl = pl.ds(i, LANES)
            ob[sl] = xb[sl] * s1 - nb[sl] * s2

    for h in in_copies(jnp.int32(0), 0):
        h.start()
    for h in in_copies(jnp.int32(1), 1):
        h.start()

    @pl.loop(jnp.int32(0), jnp.int32(NGROUP))
    def _(g):
        for slot in (0, 1):
            k = 2 * g + slot
            for h in in_copies(k, slot):
                h.wait()

            @pl.when(g > jnp.int32(0))
            def _():
                out_copy(k - 2, slot).wait()

            compute(slot)
            out_copy(k, slot).start()

            @pl.when(k + 2 < jnp.int32(NCHUNK))
            def _():
                for h in in_copies(k + 2, slot):
                    h.start()

    out_copy(jnp.int32(NCHUNK - 2), 0).wait()
    out_copy(jnp.int32(NCHUNK - 1), 1).wait()


_sc_call = functools.partial(
    pl.kernel,
    out_type=jax.ShapeDtypeStruct((SC_BATCHES * N,), jnp.float32),
    mesh=plsc.VectorSubcoreMesh(
        core_axis_name="c", subcore_axis_name="s",
        num_cores=NC, num_subcores=NS),
    scratch_types=[
        pltpu.VMEM((2 * B,), jnp.int32),
        pltpu.VMEM((1024,), jnp.float32),
        pltpu.VMEM((CHUNK,), jnp.float32),
        pltpu.VMEM((CHUNK,), jnp.float32),
        pltpu.VMEM((CHUNK,), jnp.float32),
        pltpu.VMEM((CHUNK,), jnp.float32),
        pltpu.VMEM((CHUNK,), jnp.float32),
        pltpu.VMEM((CHUNK,), jnp.float32),
        pltpu.SemaphoreType.DMA,
        pltpu.SemaphoreType.DMA,
        pltpu.SemaphoreType.DMA,
        pltpu.SemaphoreType.DMA,
    ],
)(_sc_body)

NROW = N // 128
TROW = 512


def _tc_body(ts_ref, ac_ref, noise_ref, xt_ref, out_ref):
    b = pl.program_id(0) + SC_BATCHES
    t = ts_ref[2 * b] | ts_ref[2 * b + 1]
    a = ac_ref[t]
    s1 = lax.rsqrt(a)
    s2 = jnp.sqrt(jnp.float32(1.0) - a) * s1
    out_ref[...] = xt_ref[...] * s1 - noise_ref[...] * s2


def _tc_index_map(b, j):
    return (b + jnp.int32(SC_BATCHES), j, jnp.int32(0))


if SC_BATCHES < B:
    _tc_call = pl.pallas_call(
        _tc_body,
        grid=(B - SC_BATCHES, NROW // TROW),
        in_specs=[
            pl.BlockSpec((2 * B,), lambda b, j: (jnp.int32(0),),
                         memory_space=pltpu.SMEM),
            pl.BlockSpec((AC_LEN,), lambda b, j: (jnp.int32(0),),
                         memory_space=pltpu.SMEM),
            pl.BlockSpec((1, TROW, 128), _tc_index_map),
            pl.BlockSpec((1, TROW, 128), _tc_index_map),
        ],
        out_specs=pl.BlockSpec((1, TROW, 128), _tc_index_map),
        out_shape=jax.ShapeDtypeStruct((B, NROW, 128), jnp.float32),
    )


def kernel(noise, xt, timestep, alphas_cumprod):
    ts32 = lax.bitcast_convert_type(timestep, jnp.int32).reshape(2 * B)
    ac = alphas_cumprod.astype(jnp.float32)

    sc_out = _sc_call(noise.reshape(TOTAL), xt.reshape(TOTAL), ts32, ac)
    if SC_BATCHES == B:
        return sc_out.reshape(B, C, H, W)
    tc_out = _tc_call(ts32, ac, noise.reshape(B, NROW, 128),
                      xt.reshape(B, NROW, 128))
    out = lax.dynamic_update_slice(
        tc_out, sc_out.reshape(SC_BATCHES, NROW, 128), (0, 0, 0))
    return out.reshape(B, C, H, W)
